# baseline (speedup 1.0000x reference)
import jax
import jax.numpy as jnp
from jax import lax
from jax.experimental import pallas as pl
from jax.experimental.pallas import tpu as pltpu
from jax.experimental.pallas import tpu_sc as plsc

NUM_FACTORS = 20
BATCH = 16384
LANES = 16
NUM_CORES = 2
NUM_SUBCORES = 16
NUM_WORKERS = NUM_CORES * NUM_SUBCORES
BPW = BATCH // NUM_WORKERS
HALF = BPW // 2
HGROUPS = HALF // LANES


def _rsqrt(t):
    i = plsc.bitcast(t, jnp.int32)
    i = jnp.int32(0x5F3759DF) - (i >> 1)
    y = plsc.bitcast(i, jnp.float32)
    for _ in range(3):
        y = y * (jnp.float32(1.5) - jnp.float32(0.5) * t * y * y)
    return y


def _body(users_hbm, movies_hbm, ut_hbm, mt_hbm, out_hbm,
          idx_u, idx_m, u_rows, m_rows, out_v, sem_u, sem_m):
    wid = lax.axis_index("s") * NUM_CORES + lax.axis_index("c")
    base = wid * BPW
    pltpu.sync_copy(users_hbm.at[pl.ds(base, BPW)], idx_u)
    pltpu.sync_copy(movies_hbm.at[pl.ds(base, BPW)], idx_m)

    lane = lax.iota(jnp.int32, LANES)

    for p in range(2):
        off = p * HALF

        def issue(g, carry):
            iu = idx_u[pl.ds(off + g * LANES, LANES)]
            im = idx_m[pl.ds(off + g * LANES, LANES)]
            for l in range(LANES):
                j = g * LANES + l
                pltpu.async_copy(
                    ut_hbm.at[pl.ds(iu[l], 1), :],
                    u_rows.at[pl.ds(j, 1), :], sem_u)
                pltpu.async_copy(
                    mt_hbm.at[pl.ds(im[l], 1), :],
                    m_rows.at[pl.ds(j, 1), :], sem_m)
            return carry

        lax.fori_loop(0, HGROUPS, issue, 0)

        pltpu.make_async_copy(
            ut_hbm.at[pl.ds(0, HALF), :], u_rows, sem_u).wait()
        pltpu.make_async_copy(
            mt_hbm.at[pl.ds(0, HALF), :], m_rows, sem_m).wait()

        def group(g, carry):
            rows = g * LANES + lane
            dot = jnp.zeros((LANES,), jnp.float32)
            uu = jnp.zeros((LANES,), jnp.float32)
            mm = jnp.zeros((LANES,), jnp.float32)
            for d in range(NUM_FACTORS):
                cols = jnp.full((LANES,), d, jnp.int32)
                uc = plsc.load_gather(u_rows, [rows, cols])
                mc = plsc.load_gather(m_rows, [rows, cols])
                dot = dot + uc * mc
                uu = uu + uc * uc
                mm = mm + mc * mc
            t = uu * mm
            s = t * _rsqrt(t)
            denom = jnp.maximum(s, jnp.float32(1e-8))
            out_v[pl.ds(off + g * LANES, LANES)] = dot / denom
            return carry

        lax.fori_loop(0, HGROUPS, group, 0)

    pltpu.sync_copy(out_v, out_hbm.at[pl.ds(base, BPW)])


@jax.jit
def _cosine_lookup(users, movies, user_table, movie_table):
    mesh = plsc.VectorSubcoreMesh(core_axis_name="c", subcore_axis_name="s")
    return pl.kernel(
        _body,
        mesh=mesh,
        out_type=jax.ShapeDtypeStruct((BATCH,), jnp.float32),
        scratch_types=[
            pltpu.VMEM((BPW,), jnp.int32),
            pltpu.VMEM((BPW,), jnp.int32),
            pltpu.VMEM((HALF, NUM_FACTORS), jnp.float32),
            pltpu.VMEM((HALF, NUM_FACTORS), jnp.float32),
            pltpu.VMEM((BPW,), jnp.float32),
            pltpu.SemaphoreType.DMA,
            pltpu.SemaphoreType.DMA,
        ],
        compiler_params=pltpu.CompilerParams(
            needs_layout_passes=False, use_tc_tiling_on_sc=True),
    )(users, movies, user_table, movie_table)


def kernel(users, movies, user_table, movie_table):
    return _cosine_lookup(users.astype(jnp.int32), movies.astype(jnp.int32),
                          user_table, movie_table)


# speedup vs baseline: 1.0027x; 1.0027x over previous
import jax
import jax.numpy as jnp
from jax import lax
from jax.experimental import pallas as pl
from jax.experimental.pallas import tpu as pltpu
from jax.experimental.pallas import tpu_sc as plsc

NUM_FACTORS = 20
BATCH = 16384
LANES = 16
NUM_CORES = 2
NUM_SUBCORES = 16
NUM_WORKERS = NUM_CORES * NUM_SUBCORES
BPW = BATCH // NUM_WORKERS
HALF = BPW // 2
HGROUPS = HALF // LANES
NSEM = 4


def _rsqrt(t):
    i = plsc.bitcast(t, jnp.int32)
    i = jnp.int32(0x5F3759DF) - (i >> 1)
    y = plsc.bitcast(i, jnp.float32)
    for _ in range(3):
        y = y * (jnp.float32(1.5) - jnp.float32(0.5) * t * y * y)
    return y


def _body(users_hbm, movies_hbm, ut_hbm, mt_hbm, out_hbm,
          idx_u, idx_m, u_rows, m_rows, out_v, sems_u, sems_m):
    wid = lax.axis_index("s") * NUM_CORES + lax.axis_index("c")
    base = wid * BPW
    pltpu.sync_copy(users_hbm.at[pl.ds(base, BPW)], idx_u)
    pltpu.sync_copy(movies_hbm.at[pl.ds(base, BPW)], idx_m)

    lane = lax.iota(jnp.int32, LANES)

    for p in range(2):
        off = p * HALF

        def issue(g, carry):
            iu = idx_u[pl.ds(off + g * LANES, LANES)]
            im = idx_m[pl.ds(off + g * LANES, LANES)]
            for l in range(LANES):
                j = g * LANES + l
                pltpu.async_copy(
                    ut_hbm.at[pl.ds(iu[l], 1), :],
                    u_rows.at[pl.ds(j, 1), :], sems_u.at[l % NSEM])
                pltpu.async_copy(
                    mt_hbm.at[pl.ds(im[l], 1), :],
                    m_rows.at[pl.ds(j, 1), :], sems_m.at[l % NSEM])
            return carry

        lax.fori_loop(0, HGROUPS, issue, 0)

        per_sem = HALF // NSEM
        for q in range(NSEM):
            pltpu.make_async_copy(
                ut_hbm.at[pl.ds(0, per_sem), :],
                u_rows.at[pl.ds(0, per_sem), :], sems_u.at[q]).wait()
            pltpu.make_async_copy(
                mt_hbm.at[pl.ds(0, per_sem), :],
                m_rows.at[pl.ds(0, per_sem), :], sems_m.at[q]).wait()

        def group(g, carry):
            rows = g * LANES + lane
            dot = jnp.zeros((LANES,), jnp.float32)
            uu = jnp.zeros((LANES,), jnp.float32)
            mm = jnp.zeros((LANES,), jnp.float32)
            for d in range(NUM_FACTORS):
                cols = jnp.full((LANES,), d, jnp.int32)
                uc = plsc.load_gather(u_rows, [rows, cols])
                mc = plsc.load_gather(m_rows, [rows, cols])
                dot = dot + uc * mc
                uu = uu + uc * uc
                mm = mm + mc * mc
            t = uu * mm
            s = t * _rsqrt(t)
            denom = jnp.maximum(s, jnp.float32(1e-8))
            out_v[pl.ds(off + g * LANES, LANES)] = dot / denom
            return carry

        lax.fori_loop(0, HGROUPS, group, 0)

    pltpu.sync_copy(out_v, out_hbm.at[pl.ds(base, BPW)])


@jax.jit
def _cosine_lookup(users, movies, user_table, movie_table):
    mesh = plsc.VectorSubcoreMesh(core_axis_name="c", subcore_axis_name="s")
    return pl.kernel(
        _body,
        mesh=mesh,
        out_type=jax.ShapeDtypeStruct((BATCH,), jnp.float32),
        scratch_types=[
            pltpu.VMEM((BPW,), jnp.int32),
            pltpu.VMEM((BPW,), jnp.int32),
            pltpu.VMEM((HALF, NUM_FACTORS), jnp.float32),
            pltpu.VMEM((HALF, NUM_FACTORS), jnp.float32),
            pltpu.VMEM((BPW,), jnp.float32),
            pltpu.SemaphoreType.DMA((NSEM,)),
            pltpu.SemaphoreType.DMA((NSEM,)),
        ],
        compiler_params=pltpu.CompilerParams(
            needs_layout_passes=False, use_tc_tiling_on_sc=True),
    )(users, movies, user_table, movie_table)


def kernel(users, movies, user_table, movie_table):
    return _cosine_lookup(users.astype(jnp.int32), movies.astype(jnp.int32),
                          user_table, movie_table)
